# baseline (speedup 1.0000x reference)
import numpy as np
import jax
import jax.numpy as jnp
from jax.experimental import pallas as pl


def _interp_matrix(n_in, n_out):
    xs = np.linspace(0.0, n_in - 1.0, n_out, dtype=np.float32)
    x0 = np.floor(xs).astype(np.int32)
    x1 = np.minimum(x0 + 1, n_in - 1)
    wx = (xs - x0.astype(np.float32)).astype(np.float32)
    W = np.zeros((n_out, n_in), dtype=np.float32)
    W[np.arange(n_out), x0] += 1.0 - wx
    W[np.arange(n_out), x1] += wx
    return W


def _banded_rows(W, br):
    n_out, n_in = W.shape
    nbands = n_out // br
    spans = []
    for k in range(nbands):
        cols = np.nonzero(W[k * br:(k + 1) * br].any(axis=0))[0]
        lo = 8 * (int(cols.min()) // 8)
        spans.append((lo, int(cols.max())))
    kw = max(hi - lo + 1 for lo, hi in spans)
    kw = 8 * ((kw + 7) // 8)
    starts = []
    bands = np.zeros((n_out, kw), dtype=np.float32)
    for k, (lo, hi) in enumerate(spans):
        st = min(lo, n_in - kw)
        starts.append(st)
        bands[k * br:(k + 1) * br, :] = W[k * br:(k + 1) * br, st:st + kw]
    return bands, starts, kw


def _make_body(num_classes, br, kw, starts, precision):
    nbands = len(starts)

    def _body(seg_ref, conf_ref, gb_ref, sem_ref, wyb_ref, wxt_ref,
              num_ref, den_ref):
        bi = pl.program_id(0)
        wxt = wxt_ref[...]

        dot = lambda a, b: jnp.dot(a, b, precision=precision,
                                   preferred_element_type=jnp.float32)

        cmax = [seg_ref[0, c] for c in range(num_classes)]
        while len(cmax) > 1:
            nxt = [jnp.maximum(cmax[i], cmax[i + 1])
                   for i in range(0, len(cmax) - 1, 2)]
            if len(cmax) % 2:
                nxt.append(cmax[-1])
            cmax = nxt
        cmax = cmax[0]

        ci = [dot(seg_ref[0, c] - cmax, wxt) for c in range(num_classes)]
        ln2sq = np.float32(np.log(2.0) ** 2)
        cci = dot(conf_ref[0] * ln2sq, wxt)

        pnum = jnp.zeros((), jnp.float32)
        pden = jnp.zeros((), jnp.float32)
        for k in range(nbands):
            st = starts[k]
            wyk = wyb_ref[k * br:(k + 1) * br, :]
            labels = sem_ref[0, k * br:(k + 1) * br, :]
            vf = (gb_ref[0, k * br:(k + 1) * br, :] == 0).astype(jnp.float32)
            conf_up = dot(wyk, cci[st:st + kw, :])

            s = None
            sel = None
            for c in range(num_classes):
                v = dot(wyk, ci[c][st:st + kw, :])
                e = jnp.exp2(v)
                slc = jnp.where(labels == c, v, 0.0)
                s = e if s is None else s + e
                sel = slc if sel is None else sel + slc

            nll = jnp.log2(s) - sel
            pnum = pnum + jnp.sum(nll * conf_up * vf)
            pden = pden + jnp.sum(vf)

        pnum = pnum.reshape(1, 1)
        pden = pden.reshape(1, 1)

        @pl.when(bi == 0)
        def _():
            num_ref[...] = pnum
            den_ref[...] = pden

        @pl.when(bi != 0)
        def _():
            num_ref[...] = num_ref[...] + pnum
            den_ref[...] = den_ref[...] + pden

    return _body


def kernel(seg_edge, seg_body, contrast_logits, contrast_target,
           confidence, target, gt_boundary, sem_gt):
    b, nc, h, w = seg_body.shape
    hg, wg = sem_gt.shape[1], sem_gt.shape[2]
    br = 16

    log2e = np.float32(np.log2(np.e))
    wyb_np, starts, kw = _banded_rows(_interp_matrix(h, hg), br)
    wyb = jnp.asarray(wyb_np)
    wxt = jnp.asarray(_interp_matrix(w, wg).T * log2e)
    gb = gt_boundary.astype(jnp.int32)

    num, den = pl.pallas_call(
        _make_body(nc, br, kw, starts, jax.lax.Precision.DEFAULT),
        grid=(b,),
        in_specs=[
            pl.BlockSpec((1, nc, h, w), lambda i: (i, 0, 0, 0)),
            pl.BlockSpec((1, h, w), lambda i: (i, 0, 0)),
            pl.BlockSpec((1, hg, wg), lambda i: (i, 0, 0)),
            pl.BlockSpec((1, hg, wg), lambda i: (i, 0, 0)),
            pl.BlockSpec((hg, kw), lambda i: (0, 0)),
            pl.BlockSpec((w, wg), lambda i: (0, 0)),
        ],
        out_specs=[
            pl.BlockSpec((1, 1), lambda i: (0, 0)),
            pl.BlockSpec((1, 1), lambda i: (0, 0)),
        ],
        out_shape=[
            jax.ShapeDtypeStruct((1, 1), jnp.float32),
            jax.ShapeDtypeStruct((1, 1), jnp.float32),
        ],
    )(seg_body, confidence, gb, sem_gt, wyb, wxt)

    return num[0, 0] / jnp.maximum(den[0, 0], 1.0)


# speedup vs baseline: 1.0027x; 1.0027x over previous
import numpy as np
import jax
import jax.numpy as jnp
from jax.experimental import pallas as pl


def _interp_matrix(n_in, n_out):
    xs = np.linspace(0.0, n_in - 1.0, n_out, dtype=np.float32)
    x0 = np.floor(xs).astype(np.int32)
    x1 = np.minimum(x0 + 1, n_in - 1)
    wx = (xs - x0.astype(np.float32)).astype(np.float32)
    W = np.zeros((n_out, n_in), dtype=np.float32)
    W[np.arange(n_out), x0] += 1.0 - wx
    W[np.arange(n_out), x1] += wx
    return W


def _banded_rows(W, br):
    n_out, n_in = W.shape
    nbands = n_out // br
    spans = []
    for k in range(nbands):
        cols = np.nonzero(W[k * br:(k + 1) * br].any(axis=0))[0]
        lo = 8 * (int(cols.min()) // 8)
        spans.append((lo, int(cols.max())))
    kw = max(hi - lo + 1 for lo, hi in spans)
    kw = 8 * ((kw + 7) // 8)
    starts = []
    bands = np.zeros((n_out, kw), dtype=np.float32)
    for k, (lo, hi) in enumerate(spans):
        st = min(lo, n_in - kw)
        starts.append(st)
        bands[k * br:(k + 1) * br, :] = W[k * br:(k + 1) * br, st:st + kw]
    return bands, starts, kw


def _make_body(num_classes, br, kw, starts, precision):
    nbands = len(starts)

    def _body(seg_ref, conf_ref, gb_ref, sem_ref, wyb_ref, wxt_ref,
              num_ref, den_ref):
        bi = pl.program_id(0)
        wxt = wxt_ref[...]

        dot = lambda a, b: jnp.dot(a, b, precision=precision,
                                   preferred_element_type=jnp.float32)

        cmax = [seg_ref[0, c] for c in range(num_classes)]
        while len(cmax) > 1:
            nxt = [jnp.maximum(cmax[i], cmax[i + 1])
                   for i in range(0, len(cmax) - 1, 2)]
            if len(cmax) % 2:
                nxt.append(cmax[-1])
            cmax = nxt
        cmax = cmax[0]

        ci = [dot(seg_ref[0, c] - cmax, wxt) for c in range(num_classes)]
        ln2sq = np.float32(np.log(2.0) ** 2)
        cci = dot(conf_ref[0] * ln2sq, wxt)

        pnum = jnp.zeros((), jnp.float32)
        pden = jnp.zeros((), jnp.float32)
        for k in range(nbands):
            st = starts[k]
            wyk = wyb_ref[k * br:(k + 1) * br, :]
            labels = sem_ref[0, k * br:(k + 1) * br, :]
            vf = (gb_ref[0, k * br:(k + 1) * br, :] == 0).astype(jnp.float32)
            conf_up = dot(wyk, cci[st:st + kw, :])

            s = None
            sel = None
            for c in range(num_classes):
                v = dot(wyk, ci[c][st:st + kw, :])
                e = jnp.exp2(v)
                slc = jnp.where(labels == c, v, 0.0)
                s = e if s is None else s + e
                sel = slc if sel is None else sel + slc

            nll = jnp.log2(s) - sel
            pnum = pnum + jnp.sum(nll * conf_up * vf)
            pden = pden + jnp.sum(vf)

        pnum = pnum.reshape(1, 1)
        pden = pden.reshape(1, 1)

        @pl.when(bi == 0)
        def _():
            num_ref[...] = pnum
            den_ref[...] = pden

        @pl.when(bi != 0)
        def _():
            num_ref[...] = num_ref[...] + pnum
            den_ref[...] = den_ref[...] + pden

    return _body


def kernel(seg_edge, seg_body, contrast_logits, contrast_target,
           confidence, target, gt_boundary, sem_gt):
    b, nc, h, w = seg_body.shape
    hg, wg = sem_gt.shape[1], sem_gt.shape[2]
    br = 32

    log2e = np.float32(np.log2(np.e))
    wyb_np, starts, kw = _banded_rows(_interp_matrix(h, hg), br)
    wyb = jnp.asarray(wyb_np)
    wxt = jnp.asarray(_interp_matrix(w, wg).T * log2e)
    gb = gt_boundary.astype(jnp.int32)

    num, den = pl.pallas_call(
        _make_body(nc, br, kw, starts, jax.lax.Precision.DEFAULT),
        grid=(b,),
        in_specs=[
            pl.BlockSpec((1, nc, h, w), lambda i: (i, 0, 0, 0)),
            pl.BlockSpec((1, h, w), lambda i: (i, 0, 0)),
            pl.BlockSpec((1, hg, wg), lambda i: (i, 0, 0)),
            pl.BlockSpec((1, hg, wg), lambda i: (i, 0, 0)),
            pl.BlockSpec((hg, kw), lambda i: (0, 0)),
            pl.BlockSpec((w, wg), lambda i: (0, 0)),
        ],
        out_specs=[
            pl.BlockSpec((1, 1), lambda i: (0, 0)),
            pl.BlockSpec((1, 1), lambda i: (0, 0)),
        ],
        out_shape=[
            jax.ShapeDtypeStruct((1, 1), jnp.float32),
            jax.ShapeDtypeStruct((1, 1), jnp.float32),
        ],
    )(seg_body, confidence, gb, sem_gt, wyb, wxt)

    return num[0, 0] / jnp.maximum(den[0, 0], 1.0)
